# baseline (speedup 1.0000x reference)
import jax
import jax.numpy as jnp
from jax import lax
from jax.experimental import pallas as pl
from jax.experimental.pallas import tpu as pltpu

B, T, IDIM = 8, 2048, 80
HDIM, CDIM, TNUM = 160, 16, 10
NLOOP = HDIM // CDIM
K2_MEAN = sum((k + 1) ** 2 for k in range(NLOOP)) / NLOOP
K1_MEAN = sum((k + 1) for k in range(NLOOP)) / NLOOP

BPG = 8
GRID = B // BPG


def _loss_kernel(x_ref, thr_ref, w1_ref, b1_ref, w2_ref, b2_ref, out_ref):
    g = pl.program_id(0)
    x = x_ref[...].reshape(BPG * T, IDIM)

    h = jnp.tanh(
        lax.dot_general(x, w1_ref[...], (((1,), (0,)), ((), ())),
                        preferred_element_type=jnp.float32)
        + b1_ref[...]
    )
    e = (
        lax.dot_general(h, w2_ref[...], (((1,), (0,)), ((), ())),
                        preferred_element_type=jnp.float32)
        + b2_ref[...]
    )

    def s(a, i):
        return jnp.concatenate([a[i:], a[:i]], axis=0)

    u = s(x, 1) + s(x, 2)
    w = u + s(u, 2)
    w = w + s(w, 4)
    m = (w + s(u, 8)) * (1.0 / TNUM)

    t_idx = lax.broadcasted_iota(jnp.int32, (BPG, T, 1), 1)
    vmask = (t_idx < thr_ref[...].reshape(BPG, 1, 1)).astype(jnp.float32)
    vmask = vmask.reshape(BPG * T, 1)

    q = e * vmask
    p = m * vmask
    a_part = jnp.sum(q * q)
    b_part = jnp.sum(q * p)
    c_part = jnp.sum(p * p)
    part = K2_MEAN * a_part - 2.0 * K1_MEAN * b_part + c_part

    @pl.when(g == 0)
    def _():
        out_ref[0, 0] = 0.0

    out_ref[0, 0] += part


@jax.jit
def _run(xs_pad, ilens, W1, b1, W2, b2):
    thr = (ilens - TNUM).astype(jnp.int32).reshape(B, 1)
    out = pl.pallas_call(
        _loss_kernel,
        grid=(GRID,),
        in_specs=[
            pl.BlockSpec((BPG, T, IDIM), lambda g: (g, 0, 0)),
            pl.BlockSpec((BPG, 1), lambda g: (g, 0)),
            pl.BlockSpec((IDIM, HDIM), lambda g: (0, 0)),
            pl.BlockSpec((1, HDIM), lambda g: (0, 0)),
            pl.BlockSpec((HDIM, IDIM), lambda g: (0, 0)),
            pl.BlockSpec((1, IDIM), lambda g: (0, 0)),
        ],
        out_specs=pl.BlockSpec(memory_space=pltpu.SMEM),
        out_shape=jax.ShapeDtypeStruct((1, 1), jnp.float32),
    )(xs_pad, thr, W1, b1.reshape(1, HDIM), W2, b2.reshape(1, IDIM))
    return out[0, 0]


def kernel(xs_pad, ilens, ys_pad, W1, b1, W2, b2):
    del ys_pad
    return _run(xs_pad, ilens, W1, b1, W2, b2)


# speedup vs baseline: 1.0372x; 1.0372x over previous
import jax
import jax.numpy as jnp
from jax import lax
from jax.experimental import pallas as pl
from jax.experimental.pallas import tpu as pltpu

B, T, IDIM = 8, 2048, 80
HDIM, CDIM, TNUM = 160, 16, 10
NLOOP = HDIM // CDIM
K2_MEAN = sum((k + 1) ** 2 for k in range(NLOOP)) / NLOOP
K1_MEAN = sum((k + 1) for k in range(NLOOP)) / NLOOP

BPG = 2
GRID = B // BPG


def _loss_kernel(x_ref, thr_ref, w1_ref, b1_ref, w2_ref, b2_ref, out_ref):
    g = pl.program_id(0)
    x = x_ref[...].reshape(BPG * T, IDIM)

    h = jnp.tanh(
        lax.dot_general(x, w1_ref[...], (((1,), (0,)), ((), ())),
                        preferred_element_type=jnp.float32)
        + b1_ref[...]
    )
    e = (
        lax.dot_general(h, w2_ref[...], (((1,), (0,)), ((), ())),
                        preferred_element_type=jnp.float32)
        + b2_ref[...]
    )

    def s(a, i):
        return jnp.concatenate([a[i:], a[:i]], axis=0)

    u = s(x, 1) + s(x, 2)
    w = u + s(u, 2)
    w = w + s(w, 4)
    m = (w + s(u, 8)) * (1.0 / TNUM)

    t_idx = lax.broadcasted_iota(jnp.int32, (BPG, T, 1), 1)
    vmask = (t_idx < thr_ref[0].reshape(BPG, 1, 1)).astype(jnp.float32)
    vmask = vmask.reshape(BPG * T, 1)

    q = e * vmask
    p = m * vmask
    a_part = jnp.sum(q * q)
    b_part = jnp.sum(q * p)
    c_part = jnp.sum(p * p)
    part = K2_MEAN * a_part - 2.0 * K1_MEAN * b_part + c_part

    @pl.when(g == 0)
    def _():
        out_ref[0, 0] = 0.0

    out_ref[0, 0] += part


@jax.jit
def _run(xs_pad, ilens, W1, b1, W2, b2):
    thr = (ilens - TNUM).astype(jnp.int32).reshape(GRID, BPG, 1)
    out = pl.pallas_call(
        _loss_kernel,
        grid=(GRID,),
        in_specs=[
            pl.BlockSpec((BPG, T, IDIM), lambda g: (g, 0, 0)),
            pl.BlockSpec((1, BPG, 1), lambda g: (g, 0, 0)),
            pl.BlockSpec((IDIM, HDIM), lambda g: (0, 0)),
            pl.BlockSpec((1, HDIM), lambda g: (0, 0)),
            pl.BlockSpec((HDIM, IDIM), lambda g: (0, 0)),
            pl.BlockSpec((1, IDIM), lambda g: (0, 0)),
        ],
        out_specs=pl.BlockSpec(memory_space=pltpu.SMEM),
        out_shape=jax.ShapeDtypeStruct((1, 1), jnp.float32),
    )(xs_pad, thr, W1, b1.reshape(1, HDIM), W2, b2.reshape(1, IDIM))
    return out[0, 0]


def kernel(xs_pad, ilens, ys_pad, W1, b1, W2, b2):
    del ys_pad
    return _run(xs_pad, ilens, W1, b1, W2, b2)


# speedup vs baseline: 1.1090x; 1.0693x over previous
import jax
import jax.numpy as jnp
from jax import lax
from jax.experimental import pallas as pl
from jax.experimental.pallas import tpu as pltpu

B, T, IDIM = 8, 2048, 80
HDIM, CDIM, TNUM = 160, 16, 10
NLOOP = HDIM // CDIM
K2_MEAN = sum((k + 1) ** 2 for k in range(NLOOP)) / NLOOP
K1_MEAN = sum((k + 1) for k in range(NLOOP)) / NLOOP

HB = B // 2


def _one_seq(x, thr, w1, b1, w2, b2):
    h = jnp.tanh(
        lax.dot_general(x, w1, (((1,), (0,)), ((), ())),
                        preferred_element_type=jnp.float32)
        + b1
    )
    e = (
        lax.dot_general(h, w2, (((1,), (0,)), ((), ())),
                        preferred_element_type=jnp.float32)
        + b2
    )

    def s(a, i):
        return jnp.concatenate([a[i:], a[:i]], axis=0)

    u = s(x, 1) + s(x, 2)
    w = u + s(u, 2)
    w = w + s(w, 4)
    m = (w + s(u, 8)) * (1.0 / TNUM)

    t_idx = lax.broadcasted_iota(jnp.int32, (T, 1), 0)
    vmask = (t_idx < thr).astype(jnp.float32)

    q = e * vmask
    p = m * vmask
    a_part = jnp.sum(q * q)
    b_part = jnp.sum(q * p)
    c_part = jnp.sum(p * p)
    return K2_MEAN * a_part - 2.0 * K1_MEAN * b_part + c_part


def _loss_kernel(ilens_ref, x0_ref, x1_ref, w1_ref, b1_ref, w2_ref, b2_ref,
                 out_ref):
    g = pl.program_id(0)
    w1 = w1_ref[...]
    b1 = b1_ref[...]
    w2 = w2_ref[...]
    b2 = b2_ref[...]
    part0 = _one_seq(x0_ref[0], ilens_ref[g] - TNUM, w1, b1, w2, b2)
    part1 = _one_seq(x1_ref[0], ilens_ref[g + HB] - TNUM, w1, b1, w2, b2)

    @pl.when(g == 0)
    def _():
        out_ref[0, 0] = 0.0

    out_ref[0, 0] += part0 + part1


@jax.jit
def _run(xs_pad, ilens, W1, b1, W2, b2):
    grid_spec = pltpu.PrefetchScalarGridSpec(
        num_scalar_prefetch=1,
        grid=(HB,),
        in_specs=[
            pl.BlockSpec((1, T, IDIM), lambda g, ilens: (g, 0, 0)),
            pl.BlockSpec((1, T, IDIM), lambda g, ilens: (g + HB, 0, 0)),
            pl.BlockSpec((IDIM, HDIM), lambda g, ilens: (0, 0)),
            pl.BlockSpec((1, HDIM), lambda g, ilens: (0, 0)),
            pl.BlockSpec((HDIM, IDIM), lambda g, ilens: (0, 0)),
            pl.BlockSpec((1, IDIM), lambda g, ilens: (0, 0)),
        ],
        out_specs=pl.BlockSpec(memory_space=pltpu.SMEM),
    )
    out = pl.pallas_call(
        _loss_kernel,
        grid_spec=grid_spec,
        out_shape=jax.ShapeDtypeStruct((1, 1), jnp.float32),
    )(ilens.astype(jnp.int32), xs_pad, xs_pad,
      W1, b1.reshape(1, HDIM), W2, b2.reshape(1, IDIM))
    return out[0, 0]


def kernel(xs_pad, ilens, ys_pad, W1, b1, W2, b2):
    del ys_pad
    return _run(xs_pad, ilens, W1, b1, W2, b2)
